# Initial kernel scaffold; baseline (speedup 1.0000x reference)
import jax
import jax.numpy as jnp
from jax import lax
from jax.experimental import pallas as pl
from jax.experimental.pallas import tpu as pltpu
from jax.experimental.pallas import tpu_sc as plsc

N = 10000
E = 320000
F_IN = 128
H = 16

NC = 2
NS = 16
NW = NC * NS
EPW = E // NW
GCHUNK = 1000
SCH = 125
NGC = EPW // GCHUNK
NSC = GCHUNK // SCH
DPW = EPW // SCH
SPR = N // NS

_MESH = plsc.VectorSubcoreMesh(core_axis_name="c", subcore_axis_name="s")


def _deg_body(dst_hbm, out_hbm, dst_v, ones_v, zbuf_v, acc):
    cid = lax.axis_index("c")
    sid = lax.axis_index("s")
    wid = sid * NC + cid

    @pl.loop(0, SCH)
    def _(j):
        ones_v.at[j][...] = jnp.full((H,), 1.0, jnp.float32)

    @pl.loop(0, SPR)
    def _(j):
        zbuf_v.at[j][...] = jnp.zeros((H,), jnp.float32)

    pltpu.sync_copy(zbuf_v, acc.at[pl.ds(sid * SPR, SPR)])
    pltpu.sync_copy(dst_hbm.at[pl.ds(wid * DPW, DPW)], dst_v)
    plsc.subcore_barrier()

    @pl.loop(0, DPW)
    def _(j):
        pltpu.sync_copy(ones_v, acc.at[dst_v.at[j]], add=True)

    plsc.subcore_barrier()
    pltpu.sync_copy(acc.at[pl.ds(sid * SPR, SPR)],
                    out_hbm.at[cid, pl.ds(sid * SPR, SPR)])


def _deg_call(dst2d):
    f = pl.kernel(
        _deg_body,
        out_type=jax.ShapeDtypeStruct((NC, N, H), jnp.float32),
        mesh=_MESH,
        scratch_types=[
            pltpu.VMEM((DPW, SCH), jnp.int32),
            pltpu.VMEM((SCH, H), jnp.float32),
            pltpu.VMEM((SPR, H), jnp.float32),
            pltpu.VMEM_SHARED((N, H), jnp.float32),
        ],
    )
    return f(dst2d)


def _agg_body(hp_hbm, src_hbm, dst_hbm, out_hbm, src_v, dst_v, rows_v, sem,
              acc):
    cid = lax.axis_index("c")
    sid = lax.axis_index("s")
    wid = sid * NC + cid

    pltpu.sync_copy(hp_hbm.at[pl.ds(sid * SPR, SPR)],
                    acc.at[pl.ds(sid * SPR, SPR)])
    pltpu.sync_copy(src_hbm.at[pl.ds(wid * NGC, NGC)], src_v)
    pltpu.sync_copy(dst_hbm.at[pl.ds(wid * DPW, DPW)], dst_v)
    plsc.subcore_barrier()

    @pl.loop(0, NGC)
    def _(c):
        pltpu.async_copy(hp_hbm.at[src_v.at[c]], rows_v, sem).wait()

        @pl.loop(0, NSC)
        def _(k):
            pltpu.sync_copy(rows_v.at[pl.ds(k * SCH, SCH)],
                            acc.at[dst_v.at[c * NSC + k]], add=True)

    plsc.subcore_barrier()
    pltpu.sync_copy(acc.at[pl.ds(sid * SPR, SPR)],
                    out_hbm.at[cid, pl.ds(sid * SPR, SPR)])


def _agg_call(hp, src2d, dst2d):
    f = pl.kernel(
        _agg_body,
        out_type=jax.ShapeDtypeStruct((NC, N, H), jnp.float32),
        mesh=_MESH,
        scratch_types=[
            pltpu.VMEM((NGC, GCHUNK), jnp.int32),
            pltpu.VMEM((DPW, SCH), jnp.int32),
            pltpu.VMEM((GCHUNK, H), jnp.float32),
            pltpu.SemaphoreType.DMA,
            pltpu.VMEM_SHARED((N, H), jnp.float32),
        ],
    )
    return f(hp, src2d, dst2d)


def _mm0_kernel(x_ref, w_ref, o_ref):
    o_ref[...] = jnp.dot(x_ref[...], w_ref[...],
                         preferred_element_type=jnp.float32)


def _prep_kernel(degp_ref, h0_ref, dinv_ref, hp_ref):
    deg = degp_ref[0] + degp_ref[1] + 1.0
    dinv = lax.rsqrt(deg)
    dinv_ref[...] = dinv
    hp_ref[...] = h0_ref[...] * dinv


def _combine_kernel(p_ref, hp_ref, dinv_ref, b_ref, w_ref, o_ref):
    s = p_ref[0] + p_ref[1] - hp_ref[...]
    o = s * dinv_ref[...] + b_ref[...]
    a = jnp.where(o >= 0.0, o, 0.01 * o)
    o_ref[...] = jnp.dot(a, w_ref[...],
                         preferred_element_type=jnp.float32) * dinv_ref[...]


def _final_kernel(p_ref, hp_ref, dinv_ref, b_ref, o_ref):
    s = p_ref[0] + p_ref[1] - hp_ref[...]
    o = s * dinv_ref[...] + b_ref[...]
    o_ref[...] = jax.nn.softmax(o, axis=1)


def kernel(x, edge_index, W0, b0, W1, b1, W2, b2, W3, b3):
    src2d = edge_index[0].reshape(E // GCHUNK, GCHUNK)
    dst2d = edge_index[1].reshape(E // SCH, SCH)

    f32 = jnp.float32
    sds = jax.ShapeDtypeStruct

    h0 = pl.pallas_call(_mm0_kernel, out_shape=sds((N, H), f32))(x, W0)
    degp = _deg_call(dst2d)

    dinv, hp = pl.pallas_call(
        _prep_kernel, out_shape=(sds((N, H), f32), sds((N, H), f32))
    )(degp, h0)

    combine = pl.pallas_call(_combine_kernel, out_shape=sds((N, H), f32))
    for W, b in ((W1, b0), (W2, b1), (W3, b2)):
        p = _agg_call(hp, src2d, dst2d)
        hp = combine(p, hp, dinv, b.reshape(1, H), W)

    p = _agg_call(hp, src2d, dst2d)
    out = pl.pallas_call(_final_kernel, out_shape=sds((N, H), f32))(
        p, hp, dinv, b3.reshape(1, H))
    return out


# speedup vs baseline: 44.0409x; 44.0409x over previous
import jax
import jax.numpy as jnp
from jax import lax
from jax.experimental import pallas as pl
from jax.experimental.pallas import tpu as pltpu
from jax.experimental.pallas import tpu_sc as plsc

N = 10000
E = 320000
F_IN = 128
H = 16

NC = 2
NS = 16
NW = NC * NS
EPW = E // NW
GCHUNK = 1000
SCH = 125
NGC = EPW // GCHUNK
NSC = GCHUNK // SCH
DPW = EPW // SCH
SPB = 1000
NSTR = N // SPB

_MESH = plsc.VectorSubcoreMesh(core_axis_name="c", subcore_axis_name="s")
_SC_PARAMS = pltpu.CompilerParams(use_tc_tiling_on_sc=False)


def _deg_body(dst_hbm, out_hbm, dst_v, ones_v, zbuf_v, acc):
    cid = lax.axis_index("c")
    sid = lax.axis_index("s")
    wid = sid * NC + cid

    @pl.loop(0, SCH)
    def _(j):
        ones_v.at[j][...] = jnp.full((H,), 1.0, jnp.float32)

    @pl.loop(0, SPB)
    def _(j):
        zbuf_v.at[j][...] = jnp.zeros((H,), jnp.float32)

    @pl.when(sid < NSTR)
    def _():
        pltpu.sync_copy(zbuf_v, acc.at[pl.ds(sid * SPB, SPB)])
    pltpu.sync_copy(dst_hbm.at[wid], dst_v)
    plsc.subcore_barrier()

    @pl.loop(0, DPW)
    def _(j):
        pltpu.sync_copy(ones_v, acc.at[dst_v.at[j]], add=True)

    plsc.subcore_barrier()

    @pl.when(sid < NSTR)
    def _():
        pltpu.sync_copy(acc.at[pl.ds(sid * SPB, SPB)],
                        out_hbm.at[cid, pl.ds(sid * SPB, SPB)])


def _deg_call(dst3d):
    f = pl.kernel(
        _deg_body,
        out_type=jax.ShapeDtypeStruct((NC, N, H), jnp.float32),
        mesh=_MESH,
        compiler_params=_SC_PARAMS,
        scratch_types=[
            pltpu.VMEM((DPW, SCH), jnp.int32),
            pltpu.VMEM((SCH, H), jnp.float32),
            pltpu.VMEM((SPB, H), jnp.float32),
            pltpu.VMEM_SHARED((N, H), jnp.float32),
        ],
    )
    return f(dst3d)


def _agg_body(hp_hbm, src_hbm, dst_hbm, out_hbm, src_v, dst_v, rows_v, sem,
              acc):
    cid = lax.axis_index("c")
    sid = lax.axis_index("s")
    wid = sid * NC + cid

    @pl.when(sid < NSTR)
    def _():
        pltpu.sync_copy(hp_hbm.at[pl.ds(sid * SPB, SPB)],
                        acc.at[pl.ds(sid * SPB, SPB)])
    pltpu.sync_copy(src_hbm.at[wid], src_v)
    pltpu.sync_copy(dst_hbm.at[wid], dst_v)
    plsc.subcore_barrier()

    @pl.loop(0, NGC)
    def _(c):
        pltpu.async_copy(hp_hbm.at[src_v.at[c]], rows_v, sem).wait()

        @pl.loop(0, NSC)
        def _(k):
            pltpu.sync_copy(rows_v.at[pl.ds(k * SCH, SCH)],
                            acc.at[dst_v.at[c * NSC + k]], add=True)

    plsc.subcore_barrier()

    @pl.when(sid < NSTR)
    def _():
        pltpu.sync_copy(acc.at[pl.ds(sid * SPB, SPB)],
                        out_hbm.at[cid, pl.ds(sid * SPB, SPB)])


def _agg_call(hp, src3d, dst3d):
    f = pl.kernel(
        _agg_body,
        out_type=jax.ShapeDtypeStruct((NC, N, H), jnp.float32),
        mesh=_MESH,
        compiler_params=_SC_PARAMS,
        scratch_types=[
            pltpu.VMEM((NGC, GCHUNK), jnp.int32),
            pltpu.VMEM((DPW, SCH), jnp.int32),
            pltpu.VMEM((GCHUNK, H), jnp.float32),
            pltpu.SemaphoreType.DMA,
            pltpu.VMEM_SHARED((N, H), jnp.float32),
        ],
    )
    return f(hp, src3d, dst3d)


def _mm0_kernel(x_ref, w_ref, o_ref):
    o_ref[...] = jnp.dot(x_ref[...], w_ref[...],
                         preferred_element_type=jnp.float32)


def _prep_kernel(degp_ref, h0_ref, dinv_ref, hp_ref):
    deg = degp_ref[0] + degp_ref[1] + 1.0
    dinv = lax.rsqrt(deg)
    dinv_ref[...] = dinv
    hp_ref[...] = h0_ref[...] * dinv


def _combine_kernel(p_ref, hp_ref, dinv_ref, b_ref, w_ref, o_ref):
    s = p_ref[0] + p_ref[1] - hp_ref[...]
    o = s * dinv_ref[...] + b_ref[...]
    a = jnp.where(o >= 0.0, o, 0.01 * o)
    o_ref[...] = jnp.dot(a, w_ref[...],
                         preferred_element_type=jnp.float32) * dinv_ref[...]


def _final_kernel(p_ref, hp_ref, dinv_ref, b_ref, o_ref):
    s = p_ref[0] + p_ref[1] - hp_ref[...]
    o = s * dinv_ref[...] + b_ref[...]
    o_ref[...] = jax.nn.softmax(o, axis=1)


def kernel(x, edge_index, W0, b0, W1, b1, W2, b2, W3, b3):
    src3d = edge_index[0].reshape(NW, NGC, GCHUNK)
    dst3d = edge_index[1].reshape(NW, DPW, SCH)

    f32 = jnp.float32
    sds = jax.ShapeDtypeStruct

    h0 = pl.pallas_call(_mm0_kernel, out_shape=sds((N, H), f32))(x, W0)
    degp = _deg_call(dst3d)

    dinv, hp = pl.pallas_call(
        _prep_kernel, out_shape=(sds((N, H), f32), sds((N, H), f32))
    )(degp, h0)

    combine = pl.pallas_call(_combine_kernel, out_shape=sds((N, H), f32))
    for W, b in ((W1, b0), (W2, b1), (W3, b2)):
        p = _agg_call(hp, src3d, dst3d)
        hp = combine(p, hp, dinv, b.reshape(1, H), W)

    p = _agg_call(hp, src3d, dst3d)
    out = pl.pallas_call(_final_kernel, out_shape=sds((N, H), f32))(
        p, hp, dinv, b3.reshape(1, H))
    return out


# speedup vs baseline: 51.8084x; 1.1764x over previous
import jax
import jax.numpy as jnp
from jax import lax
from jax.experimental import pallas as pl
from jax.experimental.pallas import tpu as pltpu
from jax.experimental.pallas import tpu_sc as plsc

N = 10000
E = 320000
F_IN = 128
H = 16

NC = 2
NS = 16
NW = NC * NS
EPW = E // NW
GCHUNK = 2000
SCH = 125
NGC = EPW // GCHUNK
NSC = GCHUNK // SCH
DPW = EPW // SCH
SPB = 1000
NSTR = N // SPB

_MESH = plsc.VectorSubcoreMesh(core_axis_name="c", subcore_axis_name="s")
_SC_PARAMS = pltpu.CompilerParams(use_tc_tiling_on_sc=False)


def _deg_body(dst_hbm, out_hbm, dst_v, ones_v, zbuf_v, sem_s, acc):
    cid = lax.axis_index("c")
    sid = lax.axis_index("s")
    wid = sid * NC + cid

    @pl.loop(0, SCH)
    def _(j):
        ones_v.at[j][...] = jnp.full((H,), 1.0, jnp.float32)

    @pl.loop(0, SPB)
    def _(j):
        zbuf_v.at[j][...] = jnp.zeros((H,), jnp.float32)

    @pl.when(sid < NSTR)
    def _():
        pltpu.sync_copy(zbuf_v, acc.at[pl.ds(sid * SPB, SPB)])
    pltpu.sync_copy(dst_hbm.at[wid], dst_v)
    plsc.subcore_barrier()

    handles = []
    for j in range(DPW):
        handles.append(
            pltpu.async_copy(ones_v, acc.at[dst_v.at[j]], sem_s, add=True))
        if j >= 8:
            handles[j - 8].wait()
    for h in handles[DPW - 8:]:
        h.wait()

    plsc.subcore_barrier()

    @pl.when(sid < NSTR)
    def _():
        pltpu.sync_copy(acc.at[pl.ds(sid * SPB, SPB)],
                        out_hbm.at[cid, pl.ds(sid * SPB, SPB)])


def _deg_call(dst3d):
    f = pl.kernel(
        _deg_body,
        out_type=jax.ShapeDtypeStruct((NC, N, H), jnp.float32),
        mesh=_MESH,
        compiler_params=_SC_PARAMS,
        scratch_types=[
            pltpu.VMEM((DPW, SCH), jnp.int32),
            pltpu.VMEM((SCH, H), jnp.float32),
            pltpu.VMEM((SPB, H), jnp.float32),
            pltpu.SemaphoreType.DMA,
            pltpu.VMEM_SHARED((N, H), jnp.float32),
        ],
    )
    return f(dst3d)


def _agg_body(hp_hbm, src_hbm, dst_hbm, out_hbm, src_v, dst_v, rows_v,
              sem_g0, sem_g1, sem_s0, sem_s1, acc):
    cid = lax.axis_index("c")
    sid = lax.axis_index("s")
    wid = sid * NC + cid

    @pl.when(sid < NSTR)
    def _():
        pltpu.sync_copy(hp_hbm.at[pl.ds(sid * SPB, SPB)],
                        acc.at[pl.ds(sid * SPB, SPB)])
    pltpu.sync_copy(src_hbm.at[wid], src_v)
    pltpu.sync_copy(dst_hbm.at[wid], dst_v)
    plsc.subcore_barrier()

    sem_g = (sem_g0, sem_g1)
    sem_s = (sem_s0, sem_s1)
    gh = [None] * NGC
    sh = [[None] * NSC for _ in range(NGC)]
    gh[0] = pltpu.async_copy(hp_hbm.at[src_v.at[0]], rows_v.at[0], sem_g[0])
    for c in range(NGC):
        b = c & 1
        gh[c].wait()
        if c + 1 < NGC:
            if c >= 1:
                for k in range(NSC):
                    sh[c - 1][k].wait()
            nb = (c + 1) & 1
            gh[c + 1] = pltpu.async_copy(hp_hbm.at[src_v.at[c + 1]],
                                         rows_v.at[nb], sem_g[nb])
        for k in range(NSC):
            sh[c][k] = pltpu.async_copy(
                rows_v.at[b, pl.ds(k * SCH, SCH)],
                acc.at[dst_v.at[c * NSC + k]], sem_s[b], add=True)
    for k in range(NSC):
        sh[NGC - 2][k].wait()
        sh[NGC - 1][k].wait()

    plsc.subcore_barrier()

    @pl.when(sid < NSTR)
    def _():
        pltpu.sync_copy(acc.at[pl.ds(sid * SPB, SPB)],
                        out_hbm.at[cid, pl.ds(sid * SPB, SPB)])


def _agg_call(hp, src3d, dst3d):
    f = pl.kernel(
        _agg_body,
        out_type=jax.ShapeDtypeStruct((NC, N, H), jnp.float32),
        mesh=_MESH,
        compiler_params=_SC_PARAMS,
        scratch_types=[
            pltpu.VMEM((NGC, GCHUNK), jnp.int32),
            pltpu.VMEM((DPW, SCH), jnp.int32),
            pltpu.VMEM((2, GCHUNK, H), jnp.float32),
            pltpu.SemaphoreType.DMA,
            pltpu.SemaphoreType.DMA,
            pltpu.SemaphoreType.DMA,
            pltpu.SemaphoreType.DMA,
            pltpu.VMEM_SHARED((N, H), jnp.float32),
        ],
    )
    return f(hp, src3d, dst3d)


def _mm0_kernel(x_ref, w_ref, o_ref):
    o_ref[...] = jnp.dot(x_ref[...], w_ref[...],
                         preferred_element_type=jnp.float32)


def _prep_kernel(degp_ref, h0_ref, dinv_ref, hp_ref):
    deg = degp_ref[0] + degp_ref[1] + 1.0
    dinv = lax.rsqrt(deg)
    dinv_ref[...] = dinv
    hp_ref[...] = h0_ref[...] * dinv


def _combine_kernel(p_ref, hp_ref, dinv_ref, b_ref, w_ref, o_ref):
    s = p_ref[0] + p_ref[1] - hp_ref[...]
    o = s * dinv_ref[...] + b_ref[...]
    a = jnp.where(o >= 0.0, o, 0.01 * o)
    o_ref[...] = jnp.dot(a, w_ref[...],
                         preferred_element_type=jnp.float32) * dinv_ref[...]


def _final_kernel(p_ref, hp_ref, dinv_ref, b_ref, o_ref):
    s = p_ref[0] + p_ref[1] - hp_ref[...]
    o = s * dinv_ref[...] + b_ref[...]
    o_ref[...] = jax.nn.softmax(o, axis=1)


def kernel(x, edge_index, W0, b0, W1, b1, W2, b2, W3, b3):
    src3d = edge_index[0].reshape(NW, NGC, GCHUNK)
    dst3d = edge_index[1].reshape(NW, DPW, SCH)

    f32 = jnp.float32
    sds = jax.ShapeDtypeStruct

    h0 = pl.pallas_call(_mm0_kernel, out_shape=sds((N, H), f32))(x, W0)
    degp = _deg_call(dst3d)

    dinv, hp = pl.pallas_call(
        _prep_kernel, out_shape=(sds((N, H), f32), sds((N, H), f32))
    )(degp, h0)

    combine = pl.pallas_call(_combine_kernel, out_shape=sds((N, H), f32))
    for W, b in ((W1, b0), (W2, b1), (W3, b2)):
        p = _agg_call(hp, src3d, dst3d)
        hp = combine(p, hp, dinv, b.reshape(1, H), W)

    p = _agg_call(hp, src3d, dst3d)
    out = pl.pallas_call(_final_kernel, out_shape=sds((N, H), f32))(
        p, hp, dinv, b3.reshape(1, H))
    return out


# speedup vs baseline: 70.0195x; 1.3515x over previous
import jax
import jax.numpy as jnp
from jax import lax
from jax.experimental import pallas as pl
from jax.experimental.pallas import tpu as pltpu
from jax.experimental.pallas import tpu_sc as plsc

N = 10000
E = 320000
F_IN = 128
H = 16

NC = 2
NS = 16
NW = NC * NS
EPW = E // NW
GCHUNK = 2000
SCH = 125
NGC = EPW // GCHUNK
NSC = GCHUNK // SCH
DPW = EPW // SCH
SPB = 1000
NSTR = N // SPB

_MESH = plsc.VectorSubcoreMesh(core_axis_name="c", subcore_axis_name="s")
_SC_PARAMS = pltpu.CompilerParams(use_tc_tiling_on_sc=False)


def _deg_body(dst_hbm, out_hbm, dst_v, ones_v, zbuf_v, sem_s, acc):
    cid = lax.axis_index("c")
    sid = lax.axis_index("s")
    wid = sid * NC + cid

    @pl.loop(0, SCH)
    def _(j):
        ones_v.at[j][...] = jnp.full((H,), 1.0, jnp.float32)

    @pl.loop(0, SPB)
    def _(j):
        zbuf_v.at[j][...] = jnp.zeros((H,), jnp.float32)

    @pl.when(sid < NSTR)
    def _():
        pltpu.sync_copy(zbuf_v, acc.at[pl.ds(sid * SPB, SPB)])
    pltpu.sync_copy(dst_hbm.at[wid], dst_v)
    plsc.subcore_barrier()

    handles = []
    for j in range(DPW):
        handles.append(
            pltpu.async_copy(ones_v, acc.at[dst_v.at[j]], sem_s, add=True))
        if j >= 8:
            handles[j - 8].wait()
    for h in handles[DPW - 8:]:
        h.wait()

    plsc.subcore_barrier()

    @pl.when(sid < NSTR)
    def _():
        pltpu.sync_copy(acc.at[pl.ds(sid * SPB, SPB)],
                        out_hbm.at[cid, pl.ds(sid * SPB, SPB)])


def _deg_call(dst3d):
    f = pl.kernel(
        _deg_body,
        out_type=jax.ShapeDtypeStruct((NC, N, H), jnp.float32),
        mesh=_MESH,
        compiler_params=_SC_PARAMS,
        scratch_types=[
            pltpu.VMEM((DPW, SCH), jnp.int32),
            pltpu.VMEM((SCH, H), jnp.float32),
            pltpu.VMEM((SPB, H), jnp.float32),
            pltpu.SemaphoreType.DMA,
            pltpu.VMEM_SHARED((N, H), jnp.float32),
        ],
    )
    return f(dst3d)


def _agg_body(hp_hbm, src_hbm, dst_hbm, out_hbm, src_v, dst_v, rows_v,
              sem_g0, sem_g1, sem_s0, sem_s1, acc):
    cid = lax.axis_index("c")
    sid = lax.axis_index("s")
    wid = sid * NC + cid

    @pl.when(sid < NSTR)
    def _():
        pltpu.sync_copy(hp_hbm.at[pl.ds(sid * SPB, SPB)],
                        acc.at[pl.ds(sid * SPB, SPB)])
    pltpu.sync_copy(src_hbm.at[wid], src_v)
    pltpu.sync_copy(dst_hbm.at[wid], dst_v)
    plsc.subcore_barrier()

    sem_g = (sem_g0, sem_g1)
    sem_s = (sem_s0, sem_s1)
    gh = [None] * NGC
    sh = [[None] * NSC for _ in range(NGC)]
    gh[0] = pltpu.async_copy(hp_hbm.at[src_v.at[0]], rows_v.at[0], sem_g[0])
    for c in range(NGC):
        b = c & 1
        gh[c].wait()
        if c + 1 < NGC:
            if c >= 1:
                for k in range(NSC):
                    sh[c - 1][k].wait()
            nb = (c + 1) & 1
            gh[c + 1] = pltpu.async_copy(hp_hbm.at[src_v.at[c + 1]],
                                         rows_v.at[nb], sem_g[nb])
        for k in range(NSC):
            sh[c][k] = pltpu.async_copy(
                rows_v.at[b, pl.ds(k * SCH, SCH)],
                acc.at[dst_v.at[c * NSC + k]], sem_s[b], add=True)
    for k in range(NSC):
        sh[NGC - 2][k].wait()
        sh[NGC - 1][k].wait()

    plsc.subcore_barrier()

    @pl.when(sid < NSTR)
    def _():
        pltpu.sync_copy(acc.at[pl.ds(sid * SPB, SPB)],
                        out_hbm.at[cid, pl.ds(sid * SPB, SPB)])


def _agg_call(hp, src3d, dst3d):
    f = pl.kernel(
        _agg_body,
        out_type=jax.ShapeDtypeStruct((NC, N, H), jnp.float32),
        mesh=_MESH,
        compiler_params=_SC_PARAMS,
        scratch_types=[
            pltpu.VMEM((NGC, GCHUNK), jnp.int32),
            pltpu.VMEM((DPW, SCH), jnp.int32),
            pltpu.VMEM((2, GCHUNK, H), jnp.float32),
            pltpu.SemaphoreType.DMA,
            pltpu.SemaphoreType.DMA,
            pltpu.SemaphoreType.DMA,
            pltpu.SemaphoreType.DMA,
            pltpu.VMEM_SHARED((N, H), jnp.float32),
        ],
    )
    return f(hp, src3d, dst3d)


NP8 = N // 8


def _mm0_kernel(x_ref, w_ref, o_ref):
    o_ref[...] = jnp.dot(x_ref[...], w_ref[...],
                         preferred_element_type=jnp.float32)


def _prep_kernel(degp_ref, h0_ref, dinv_ref, hp_ref):
    deg = degp_ref[0] + degp_ref[1] + 1.0
    dinv = lax.rsqrt(deg)
    dinv_ref[...] = dinv
    hp_ref[...] = h0_ref[...] * dinv


def _combine_kernel(p_ref, hp_ref, dinv_ref, b_ref, w_ref, o_ref):
    s = p_ref[0] + p_ref[1] - hp_ref[...]
    o = s * dinv_ref[...] + b_ref[...]
    a = jnp.where(o >= 0.0, o, 0.01 * o)
    o_ref[...] = jnp.dot(a, w_ref[...],
                         preferred_element_type=jnp.float32) * dinv_ref[...]


def _logits_kernel(p_ref, hp_ref, dinv_ref, b_ref, o_ref):
    s = p_ref[0] + p_ref[1] - hp_ref[...]
    o_ref[...] = s * dinv_ref[...] + b_ref[...]


def _softmax_kernel(x_ref, o_ref):
    o_ref[...] = jax.nn.softmax(x_ref[...], axis=1)


def kernel(x, edge_index, W0, b0, W1, b1, W2, b2, W3, b3):
    src3d = edge_index[0].reshape(NW, NGC, GCHUNK)
    dst3d = edge_index[1].reshape(NW, DPW, SCH)

    f32 = jnp.float32
    sds = jax.ShapeDtypeStruct
    eye8 = jnp.eye(8, dtype=f32)

    h0p = pl.pallas_call(_mm0_kernel, out_shape=sds((NP8, 128), f32))(
        x.reshape(NP8, 8 * F_IN), jnp.kron(eye8, W0))
    degp = _deg_call(dst3d).reshape(NC, NP8, 128)

    dinv, hp = pl.pallas_call(
        _prep_kernel, out_shape=(sds((NP8, 128), f32), sds((NP8, 128), f32))
    )(degp, h0p)

    combine = pl.pallas_call(_combine_kernel, out_shape=sds((NP8, 128), f32))
    for W, b in ((W1, b0), (W2, b1), (W3, b2)):
        p = _agg_call(hp.reshape(N, H), src3d, dst3d).reshape(NC, NP8, 128)
        hp = combine(p, hp, dinv, jnp.tile(b, 8).reshape(1, 128),
                     jnp.kron(eye8, W))

    p = _agg_call(hp.reshape(N, H), src3d, dst3d).reshape(NC, NP8, 128)
    logits = pl.pallas_call(_logits_kernel, out_shape=sds((NP8, 128), f32))(
        p, hp, dinv, jnp.tile(b3, 8).reshape(1, 128))
    out = pl.pallas_call(_softmax_kernel, out_shape=sds((N, H), f32))(
        logits.reshape(N, H))
    return out


# speedup vs baseline: 73.4259x; 1.0486x over previous
import jax
import jax.numpy as jnp
from jax import lax
from jax.experimental import pallas as pl
from jax.experimental.pallas import tpu as pltpu
from jax.experimental.pallas import tpu_sc as plsc

N = 10000
E = 320000
F_IN = 128
H = 16

NC = 2
NS = 16
NW = NC * NS
EPW = E // NW
GCHUNK = 2000
SCH = 250
NGC = EPW // GCHUNK
NSC = GCHUNK // SCH
DPW = EPW // SCH
SPB = 1000
NSTR = N // SPB

_MESH = plsc.VectorSubcoreMesh(core_axis_name="c", subcore_axis_name="s")
_SC_PARAMS = pltpu.CompilerParams(use_tc_tiling_on_sc=False)


def _deg_body(dst_hbm, out_hbm, dst_v, ones_v, zbuf_v, sem_s, acc):
    cid = lax.axis_index("c")
    sid = lax.axis_index("s")
    wid = sid * NC + cid

    @pl.loop(0, SCH)
    def _(j):
        ones_v.at[j][...] = jnp.full((H,), 1.0, jnp.float32)

    @pl.loop(0, SPB)
    def _(j):
        zbuf_v.at[j][...] = jnp.zeros((H,), jnp.float32)

    @pl.when(sid < NSTR)
    def _():
        pltpu.sync_copy(zbuf_v, acc.at[pl.ds(sid * SPB, SPB)])
    pltpu.sync_copy(dst_hbm.at[wid], dst_v)
    plsc.subcore_barrier()

    handles = []
    for j in range(DPW):
        handles.append(
            pltpu.async_copy(ones_v, acc.at[dst_v.at[j]], sem_s, add=True))
        if j >= 8:
            handles[j - 8].wait()
    for h in handles[DPW - 8:]:
        h.wait()

    plsc.subcore_barrier()

    @pl.when(sid < NSTR)
    def _():
        pltpu.sync_copy(acc.at[pl.ds(sid * SPB, SPB)],
                        out_hbm.at[cid, pl.ds(sid * SPB, SPB)])


def _deg_call(dst3d):
    f = pl.kernel(
        _deg_body,
        out_type=jax.ShapeDtypeStruct((NC, N, H), jnp.float32),
        mesh=_MESH,
        compiler_params=_SC_PARAMS,
        scratch_types=[
            pltpu.VMEM((DPW, SCH), jnp.int32),
            pltpu.VMEM((SCH, H), jnp.float32),
            pltpu.VMEM((SPB, H), jnp.float32),
            pltpu.SemaphoreType.DMA,
            pltpu.VMEM_SHARED((N, H), jnp.float32),
        ],
    )
    return f(dst3d)


def _agg_body(hp_hbm, src_hbm, dst_hbm, out_hbm, src_v, dst_v, rows_v,
              sem_g0, sem_g1, sem_g2, sem_s0, sem_s1, sem_s2, acc):
    cid = lax.axis_index("c")
    sid = lax.axis_index("s")
    wid = sid * NC + cid

    @pl.when(sid < NSTR)
    def _():
        pltpu.sync_copy(hp_hbm.at[pl.ds(sid * SPB, SPB)],
                        acc.at[pl.ds(sid * SPB, SPB)])
    pltpu.sync_copy(src_hbm.at[wid], src_v)
    pltpu.sync_copy(dst_hbm.at[wid], dst_v)
    plsc.subcore_barrier()

    NBUF = 3
    sem_g = (sem_g0, sem_g1, sem_g2)
    sem_s = (sem_s0, sem_s1, sem_s2)
    gh = [None] * NGC
    sh = [[None] * NSC for _ in range(NGC)]
    gh[0] = pltpu.async_copy(hp_hbm.at[src_v.at[0]], rows_v.at[0], sem_g0)
    gh[1] = pltpu.async_copy(hp_hbm.at[src_v.at[1]], rows_v.at[1], sem_g1)
    for c in range(NGC):
        b = c % NBUF
        gh[c].wait()
        if c + 2 < NGC:
            if c >= 1:
                for k in range(NSC):
                    sh[c - 1][k].wait()
            nb = (c + 2) % NBUF
            gh[c + 2] = pltpu.async_copy(hp_hbm.at[src_v.at[c + 2]],
                                         rows_v.at[nb], sem_g[nb])
        for k in range(NSC):
            sh[c][k] = pltpu.async_copy(
                rows_v.at[b, pl.ds(k * SCH, SCH)],
                acc.at[dst_v.at[c * NSC + k]], sem_s[b], add=True)
    for k in range(NSC):
        sh[NGC - 2][k].wait()
        sh[NGC - 1][k].wait()

    plsc.subcore_barrier()

    @pl.when(sid < NSTR)
    def _():
        pltpu.sync_copy(acc.at[pl.ds(sid * SPB, SPB)],
                        out_hbm.at[cid, pl.ds(sid * SPB, SPB)])


def _agg_call(hp, src3d, dst3d):
    f = pl.kernel(
        _agg_body,
        out_type=jax.ShapeDtypeStruct((NC, N, H), jnp.float32),
        mesh=_MESH,
        compiler_params=_SC_PARAMS,
        scratch_types=[
            pltpu.VMEM((NGC, GCHUNK), jnp.int32),
            pltpu.VMEM((DPW, SCH), jnp.int32),
            pltpu.VMEM((3, GCHUNK, H), jnp.float32),
            pltpu.SemaphoreType.DMA,
            pltpu.SemaphoreType.DMA,
            pltpu.SemaphoreType.DMA,
            pltpu.SemaphoreType.DMA,
            pltpu.SemaphoreType.DMA,
            pltpu.SemaphoreType.DMA,
            pltpu.VMEM_SHARED((N, H), jnp.float32),
        ],
    )
    return f(hp, src3d, dst3d)


NP8 = N // 8


def _mm0_kernel(x_ref, w_ref, o_ref):
    o_ref[...] = jnp.dot(x_ref[...], w_ref[...],
                         preferred_element_type=jnp.float32)


def _prep_kernel(degp_ref, h0_ref, dinv_ref, hp_ref):
    deg = degp_ref[0] + degp_ref[1] + 1.0
    dinv = lax.rsqrt(deg)
    dinv_ref[...] = dinv
    hp_ref[...] = h0_ref[...] * dinv


def _combine_kernel(p_ref, hp_ref, dinv_ref, b_ref, w_ref, o_ref):
    s = p_ref[0] + p_ref[1] - hp_ref[...]
    o = s * dinv_ref[...] + b_ref[...]
    a = jnp.where(o >= 0.0, o, 0.01 * o)
    o_ref[...] = jnp.dot(a, w_ref[...],
                         preferred_element_type=jnp.float32) * dinv_ref[...]


def _logits_kernel(p_ref, hp_ref, dinv_ref, b_ref, o_ref):
    s = p_ref[0] + p_ref[1] - hp_ref[...]
    o_ref[...] = s * dinv_ref[...] + b_ref[...]


def _softmax_kernel(x_ref, o_ref):
    o_ref[...] = jax.nn.softmax(x_ref[...], axis=1)


def kernel(x, edge_index, W0, b0, W1, b1, W2, b2, W3, b3):
    src3d = edge_index[0].reshape(NW, NGC, GCHUNK)
    dst3d = edge_index[1].reshape(NW, DPW, SCH)

    f32 = jnp.float32
    sds = jax.ShapeDtypeStruct
    eye8 = jnp.eye(8, dtype=f32)

    h0p = pl.pallas_call(_mm0_kernel, out_shape=sds((NP8, 128), f32))(
        x.reshape(NP8, 8 * F_IN), jnp.kron(eye8, W0))
    degp = _deg_call(dst3d).reshape(NC, NP8, 128)

    dinv, hp = pl.pallas_call(
        _prep_kernel, out_shape=(sds((NP8, 128), f32), sds((NP8, 128), f32))
    )(degp, h0p)

    combine = pl.pallas_call(_combine_kernel, out_shape=sds((NP8, 128), f32))
    for W, b in ((W1, b0), (W2, b1), (W3, b2)):
        p = _agg_call(hp.reshape(N, H), src3d, dst3d).reshape(NC, NP8, 128)
        hp = combine(p, hp, dinv, jnp.tile(b, 8).reshape(1, 128),
                     jnp.kron(eye8, W))

    p = _agg_call(hp.reshape(N, H), src3d, dst3d).reshape(NC, NP8, 128)
    logits = pl.pallas_call(_logits_kernel, out_shape=sds((NP8, 128), f32))(
        p, hp, dinv, jnp.tile(b3, 8).reshape(1, 128))
    out = pl.pallas_call(_softmax_kernel, out_shape=sds((N, H), f32))(
        logits.reshape(N, H))
    return out


# speedup vs baseline: 73.5628x; 1.0019x over previous
import jax
import jax.numpy as jnp
from jax import lax
from jax.experimental import pallas as pl
from jax.experimental.pallas import tpu as pltpu
from jax.experimental.pallas import tpu_sc as plsc

N = 10000
E = 320000
F_IN = 128
H = 16

NC = 2
NS = 16
NW = NC * NS
EPW = E // NW
GCHUNK = 2000
SCH = 500
NGC = EPW // GCHUNK
NSC = GCHUNK // SCH
DPW = EPW // SCH
SPB = 1000
NSTR = N // SPB

_MESH = plsc.VectorSubcoreMesh(core_axis_name="c", subcore_axis_name="s")
_SC_PARAMS = pltpu.CompilerParams(use_tc_tiling_on_sc=False)


def _deg_body(dst_hbm, out_hbm, dst_v, ones_v, zbuf_v, sem_s, acc):
    cid = lax.axis_index("c")
    sid = lax.axis_index("s")
    wid = sid * NC + cid

    @pl.loop(0, SCH)
    def _(j):
        ones_v.at[j][...] = jnp.full((H,), 1.0, jnp.float32)

    @pl.loop(0, SPB)
    def _(j):
        zbuf_v.at[j][...] = jnp.zeros((H,), jnp.float32)

    @pl.when(sid < NSTR)
    def _():
        pltpu.sync_copy(zbuf_v, acc.at[pl.ds(sid * SPB, SPB)])
    pltpu.sync_copy(dst_hbm.at[wid], dst_v)
    plsc.subcore_barrier()

    handles = []
    for j in range(DPW):
        handles.append(
            pltpu.async_copy(ones_v, acc.at[dst_v.at[j]], sem_s, add=True))
        if j >= 8:
            handles[j - 8].wait()
    for h in handles[DPW - 8:]:
        h.wait()

    plsc.subcore_barrier()

    @pl.when(sid < NSTR)
    def _():
        pltpu.sync_copy(acc.at[pl.ds(sid * SPB, SPB)],
                        out_hbm.at[cid, pl.ds(sid * SPB, SPB)])


def _deg_call(dst3d):
    f = pl.kernel(
        _deg_body,
        out_type=jax.ShapeDtypeStruct((NC, N, H), jnp.float32),
        mesh=_MESH,
        compiler_params=_SC_PARAMS,
        scratch_types=[
            pltpu.VMEM((DPW, SCH), jnp.int32),
            pltpu.VMEM((SCH, H), jnp.float32),
            pltpu.VMEM((SPB, H), jnp.float32),
            pltpu.SemaphoreType.DMA,
            pltpu.VMEM_SHARED((N, H), jnp.float32),
        ],
    )
    return f(dst3d)


def _agg_body(hp_hbm, src_hbm, dst_hbm, out_hbm, src_v, dst_v, rows_v,
              sem_g0, sem_g1, sem_g2, sem_s0, sem_s1, sem_s2, acc):
    cid = lax.axis_index("c")
    sid = lax.axis_index("s")
    wid = sid * NC + cid

    @pl.when(sid < NSTR)
    def _():
        pltpu.sync_copy(hp_hbm.at[pl.ds(sid * SPB, SPB)],
                        acc.at[pl.ds(sid * SPB, SPB)])
    pltpu.sync_copy(src_hbm.at[wid], src_v)
    pltpu.sync_copy(dst_hbm.at[wid], dst_v)
    plsc.subcore_barrier()

    NBUF = 3
    sem_g = (sem_g0, sem_g1, sem_g2)
    sem_s = (sem_s0, sem_s1, sem_s2)
    gh = [None] * NGC
    sh = [[None] * NSC for _ in range(NGC)]
    gh[0] = pltpu.async_copy(hp_hbm.at[src_v.at[0]], rows_v.at[0], sem_g0)
    gh[1] = pltpu.async_copy(hp_hbm.at[src_v.at[1]], rows_v.at[1], sem_g1)
    for c in range(NGC):
        b = c % NBUF
        gh[c].wait()
        if c + 2 < NGC:
            if c >= 1:
                for k in range(NSC):
                    sh[c - 1][k].wait()
            nb = (c + 2) % NBUF
            gh[c + 2] = pltpu.async_copy(hp_hbm.at[src_v.at[c + 2]],
                                         rows_v.at[nb], sem_g[nb])
        for k in range(NSC):
            sh[c][k] = pltpu.async_copy(
                rows_v.at[b, pl.ds(k * SCH, SCH)],
                acc.at[dst_v.at[c * NSC + k]], sem_s[b], add=True)
    for k in range(NSC):
        sh[NGC - 2][k].wait()
        sh[NGC - 1][k].wait()

    plsc.subcore_barrier()

    @pl.when(sid < NSTR)
    def _():
        pltpu.sync_copy(acc.at[pl.ds(sid * SPB, SPB)],
                        out_hbm.at[cid, pl.ds(sid * SPB, SPB)])


def _agg_call(hp, src3d, dst3d):
    f = pl.kernel(
        _agg_body,
        out_type=jax.ShapeDtypeStruct((NC, N, H), jnp.float32),
        mesh=_MESH,
        compiler_params=_SC_PARAMS,
        scratch_types=[
            pltpu.VMEM((NGC, GCHUNK), jnp.int32),
            pltpu.VMEM((DPW, SCH), jnp.int32),
            pltpu.VMEM((3, GCHUNK, H), jnp.float32),
            pltpu.SemaphoreType.DMA,
            pltpu.SemaphoreType.DMA,
            pltpu.SemaphoreType.DMA,
            pltpu.SemaphoreType.DMA,
            pltpu.SemaphoreType.DMA,
            pltpu.SemaphoreType.DMA,
            pltpu.VMEM_SHARED((N, H), jnp.float32),
        ],
    )
    return f(hp, src3d, dst3d)


NP8 = N // 8


def _mm0_kernel(x_ref, w_ref, o_ref):
    o_ref[...] = jnp.dot(x_ref[...], w_ref[...],
                         preferred_element_type=jnp.float32)


def _prep_kernel(degp_ref, h0_ref, dinv_ref, hp_ref):
    deg = degp_ref[0] + degp_ref[1] + 1.0
    dinv = lax.rsqrt(deg)
    dinv_ref[...] = dinv
    hp_ref[...] = h0_ref[...] * dinv


def _combine_kernel(p_ref, hp_ref, dinv_ref, b_ref, w_ref, o_ref):
    s = p_ref[0] + p_ref[1] - hp_ref[...]
    o = s * dinv_ref[...] + b_ref[...]
    a = jnp.where(o >= 0.0, o, 0.01 * o)
    o_ref[...] = jnp.dot(a, w_ref[...],
                         preferred_element_type=jnp.float32) * dinv_ref[...]


def _logits_kernel(p_ref, hp_ref, dinv_ref, b_ref, o_ref):
    s = p_ref[0] + p_ref[1] - hp_ref[...]
    o_ref[...] = s * dinv_ref[...] + b_ref[...]


def _softmax_kernel(x_ref, o_ref):
    o_ref[...] = jax.nn.softmax(x_ref[...], axis=1)


def kernel(x, edge_index, W0, b0, W1, b1, W2, b2, W3, b3):
    src3d = edge_index[0].reshape(NW, NGC, GCHUNK)
    dst3d = edge_index[1].reshape(NW, DPW, SCH)

    f32 = jnp.float32
    sds = jax.ShapeDtypeStruct
    eye8 = jnp.eye(8, dtype=f32)

    h0p = pl.pallas_call(_mm0_kernel, out_shape=sds((NP8, 128), f32))(
        x.reshape(NP8, 8 * F_IN), jnp.kron(eye8, W0))
    degp = _deg_call(dst3d).reshape(NC, NP8, 128)

    dinv, hp = pl.pallas_call(
        _prep_kernel, out_shape=(sds((NP8, 128), f32), sds((NP8, 128), f32))
    )(degp, h0p)

    combine = pl.pallas_call(_combine_kernel, out_shape=sds((NP8, 128), f32))
    for W, b in ((W1, b0), (W2, b1), (W3, b2)):
        p = _agg_call(hp.reshape(N, H), src3d, dst3d).reshape(NC, NP8, 128)
        hp = combine(p, hp, dinv, jnp.tile(b, 8).reshape(1, 128),
                     jnp.kron(eye8, W))

    p = _agg_call(hp.reshape(N, H), src3d, dst3d).reshape(NC, NP8, 128)
    logits = pl.pallas_call(_logits_kernel, out_shape=sds((NP8, 128), f32))(
        p, hp, dinv, jnp.tile(b3, 8).reshape(1, 128))
    out = pl.pallas_call(_softmax_kernel, out_shape=sds((N, H), f32))(
        logits.reshape(N, H))
    return out


# speedup vs baseline: 78.9166x; 1.0728x over previous
import jax
import jax.numpy as jnp
from jax import lax
from jax.experimental import pallas as pl
from jax.experimental.pallas import tpu as pltpu
from jax.experimental.pallas import tpu_sc as plsc

N = 10000
E = 320000
F_IN = 128
H = 16

NC = 2
NS = 16
NW = NC * NS
EPW = E // NW
GCHUNK = 2000
SCH = 1000
NGC = EPW // GCHUNK
NSC = GCHUNK // SCH
DPW = EPW // SCH
SPB = 1000
NSTR = N // SPB

_MESH = plsc.VectorSubcoreMesh(core_axis_name="c", subcore_axis_name="s")
_SC_PARAMS = pltpu.CompilerParams(use_tc_tiling_on_sc=False)


def _deg_body(edge_hbm, out_hbm, dst_v, ones_v, zbuf_v, sem_s, acc):
    cid = lax.axis_index("c")
    sid = lax.axis_index("s")
    wid = sid * NC + cid

    @pl.loop(0, SCH)
    def _(j):
        ones_v.at[j][...] = jnp.full((H,), 1.0, jnp.float32)

    @pl.loop(0, SPB)
    def _(j):
        zbuf_v.at[j][...] = jnp.zeros((H,), jnp.float32)

    @pl.when(sid < NSTR)
    def _():
        pltpu.sync_copy(zbuf_v, acc.at[pl.ds(sid * SPB, SPB)])
    pltpu.sync_copy(edge_hbm.at[1, pl.ds(wid * EPW, EPW)], dst_v)
    plsc.subcore_barrier()

    handles = []
    for j in range(DPW):
        handles.append(
            pltpu.async_copy(ones_v, acc.at[dst_v.at[pl.ds(j * SCH, SCH)]],
                             sem_s, add=True))
        if j >= 4:
            handles[j - 4].wait()
    for h in handles[DPW - 4:]:
        h.wait()

    plsc.subcore_barrier()

    @pl.when(sid < NSTR)
    def _():
        pltpu.sync_copy(acc.at[pl.ds(sid * SPB, SPB)],
                        out_hbm.at[cid, pl.ds(sid * SPB, SPB)])


def _deg_call(edge_index):
    f = pl.kernel(
        _deg_body,
        out_type=jax.ShapeDtypeStruct((NC, N, H), jnp.float32),
        mesh=_MESH,
        compiler_params=_SC_PARAMS,
        scratch_types=[
            pltpu.VMEM((EPW,), jnp.int32),
            pltpu.VMEM((SCH, H), jnp.float32),
            pltpu.VMEM((SPB, H), jnp.float32),
            pltpu.SemaphoreType.DMA,
            pltpu.VMEM_SHARED((N, H), jnp.float32),
        ],
    )
    return f(edge_index)


def _agg_body(hp_hbm, edge_hbm, out_hbm, src_v, dst_v, rows_v,
              sem_g0, sem_g1, sem_g2, sem_s0, sem_s1, sem_s2, acc):
    cid = lax.axis_index("c")
    sid = lax.axis_index("s")
    wid = sid * NC + cid

    @pl.when(sid < NSTR)
    def _():
        pltpu.sync_copy(hp_hbm.at[pl.ds(sid * SPB, SPB)],
                        acc.at[pl.ds(sid * SPB, SPB)])
    pltpu.sync_copy(edge_hbm.at[0, pl.ds(wid * EPW, EPW)], src_v)
    pltpu.sync_copy(edge_hbm.at[1, pl.ds(wid * EPW, EPW)], dst_v)
    plsc.subcore_barrier()

    NBUF = 3
    sem_g = (sem_g0, sem_g1, sem_g2)
    sem_s = (sem_s0, sem_s1, sem_s2)
    gh = [None] * NGC
    sh = [[None] * NSC for _ in range(NGC)]
    gh[0] = pltpu.async_copy(hp_hbm.at[src_v.at[pl.ds(0, GCHUNK)]],
                             rows_v.at[0], sem_g0)
    gh[1] = pltpu.async_copy(hp_hbm.at[src_v.at[pl.ds(GCHUNK, GCHUNK)]],
                             rows_v.at[1], sem_g1)
    for c in range(NGC):
        b = c % NBUF
        gh[c].wait()
        if c + 2 < NGC:
            if c >= 1:
                for k in range(NSC):
                    sh[c - 1][k].wait()
            nb = (c + 2) % NBUF
            gh[c + 2] = pltpu.async_copy(
                hp_hbm.at[src_v.at[pl.ds((c + 2) * GCHUNK, GCHUNK)]],
                rows_v.at[nb], sem_g[nb])
        for k in range(NSC):
            sh[c][k] = pltpu.async_copy(
                rows_v.at[b, pl.ds(k * SCH, SCH)],
                acc.at[dst_v.at[pl.ds((c * NSC + k) * SCH, SCH)]],
                sem_s[b], add=True)
    for k in range(NSC):
        sh[NGC - 2][k].wait()
        sh[NGC - 1][k].wait()

    plsc.subcore_barrier()

    @pl.when(sid < NSTR)
    def _():
        pltpu.sync_copy(acc.at[pl.ds(sid * SPB, SPB)],
                        out_hbm.at[cid, pl.ds(sid * SPB, SPB)])


def _agg_call(hp, edge_index):
    f = pl.kernel(
        _agg_body,
        out_type=jax.ShapeDtypeStruct((NC, N, H), jnp.float32),
        mesh=_MESH,
        compiler_params=_SC_PARAMS,
        scratch_types=[
            pltpu.VMEM((EPW,), jnp.int32),
            pltpu.VMEM((EPW,), jnp.int32),
            pltpu.VMEM((3, GCHUNK, H), jnp.float32),
            pltpu.SemaphoreType.DMA,
            pltpu.SemaphoreType.DMA,
            pltpu.SemaphoreType.DMA,
            pltpu.SemaphoreType.DMA,
            pltpu.SemaphoreType.DMA,
            pltpu.SemaphoreType.DMA,
            pltpu.VMEM_SHARED((N, H), jnp.float32),
        ],
    )
    return f(hp, edge_index)


NP8 = N // 8


def _mm0_kernel(x_ref, w_ref, o_ref):
    o_ref[...] = jnp.dot(x_ref[...], w_ref[...],
                         preferred_element_type=jnp.float32)


def _prep_kernel(degp_ref, h0_ref, dinv_ref, hp_ref):
    deg = degp_ref[0] + degp_ref[1] + 1.0
    dinv = lax.rsqrt(deg)
    dinv_ref[...] = dinv
    hp_ref[...] = h0_ref[...] * dinv


def _combine_kernel(p_ref, hp_ref, dinv_ref, b_ref, w_ref, o_ref):
    s = p_ref[0] + p_ref[1] - hp_ref[...]
    o = s * dinv_ref[...] + b_ref[...]
    a = jnp.where(o >= 0.0, o, 0.01 * o)
    o_ref[...] = jnp.dot(a, w_ref[...],
                         preferred_element_type=jnp.float32) * dinv_ref[...]


def _logits_kernel(p_ref, hp_ref, dinv_ref, b_ref, o_ref):
    s = p_ref[0] + p_ref[1] - hp_ref[...]
    o_ref[...] = s * dinv_ref[...] + b_ref[...]


def _softmax_kernel(x_ref, o_ref):
    o_ref[...] = jax.nn.softmax(x_ref[...], axis=1)


def kernel(x, edge_index, W0, b0, W1, b1, W2, b2, W3, b3):
    f32 = jnp.float32
    sds = jax.ShapeDtypeStruct
    eye8 = jnp.eye(8, dtype=f32)

    h0p = pl.pallas_call(_mm0_kernel, out_shape=sds((NP8, 128), f32))(
        x.reshape(NP8, 8 * F_IN), jnp.kron(eye8, W0))
    degp = _deg_call(edge_index).reshape(NC, NP8, 128)

    dinv, hp = pl.pallas_call(
        _prep_kernel, out_shape=(sds((NP8, 128), f32), sds((NP8, 128), f32))
    )(degp, h0p)

    combine = pl.pallas_call(_combine_kernel, out_shape=sds((NP8, 128), f32))
    for W, b in ((W1, b0), (W2, b1), (W3, b2)):
        p = _agg_call(hp.reshape(N, H), edge_index).reshape(NC, NP8, 128)
        hp = combine(p, hp, dinv, jnp.tile(b, 8).reshape(1, 128),
                     jnp.kron(eye8, W))

    p = _agg_call(hp.reshape(N, H), edge_index).reshape(NC, NP8, 128)
    logits = pl.pallas_call(_logits_kernel, out_shape=sds((NP8, 128), f32))(
        p, hp, dinv, jnp.tile(b3, 8).reshape(1, 128))
    out = pl.pallas_call(_softmax_kernel, out_shape=sds((N, H), f32))(
        logits.reshape(N, H))
    return out


# speedup vs baseline: 83.0661x; 1.0526x over previous
import jax
import jax.numpy as jnp
from jax import lax
from jax.experimental import pallas as pl
from jax.experimental.pallas import tpu as pltpu
from jax.experimental.pallas import tpu_sc as plsc

N = 10000
E = 320000
F_IN = 128
H = 16

NC = 2
NS = 16
NW = NC * NS
EPW = E // NW
GCHUNK = 2000
SCH = 1000
NGC = EPW // GCHUNK
NSC = GCHUNK // SCH
DPW = EPW // SCH
SPB = 1000
NSTR = N // SPB

_MESH = plsc.VectorSubcoreMesh(core_axis_name="c", subcore_axis_name="s")
_SC_PARAMS = pltpu.CompilerParams(use_tc_tiling_on_sc=False)


def _deg_body(edge_hbm, out_hbm, dst_v, ones_v, zbuf_v, sem_s, acc):
    cid = lax.axis_index("c")
    sid = lax.axis_index("s")
    wid = sid * NC + cid

    dh = pltpu.async_copy(edge_hbm.at[1, pl.ds(wid * EPW, EPW)], dst_v,
                          sem_s)

    @pl.loop(0, SCH)
    def _(j):
        ones_v.at[j][...] = jnp.full((H,), 1.0, jnp.float32)

    @pl.loop(0, SPB)
    def _(j):
        zbuf_v.at[j][...] = jnp.zeros((H,), jnp.float32)

    @pl.when(sid < NSTR)
    def _():
        pltpu.sync_copy(zbuf_v, acc.at[pl.ds(sid * SPB, SPB)])
    plsc.subcore_barrier()

    dh.wait()

    handles = []
    for j in range(DPW):
        handles.append(
            pltpu.async_copy(ones_v, acc.at[dst_v.at[pl.ds(j * SCH, SCH)]],
                             sem_s, add=True))
        if j >= 4:
            handles[j - 4].wait()
    for h in handles[DPW - 4:]:
        h.wait()

    plsc.subcore_barrier()

    @pl.when(sid < NSTR)
    def _():
        pltpu.sync_copy(acc.at[pl.ds(sid * SPB, SPB)],
                        out_hbm.at[cid, pl.ds(sid * SPB, SPB)])


def _deg_call(edge_index):
    f = pl.kernel(
        _deg_body,
        out_type=jax.ShapeDtypeStruct((NC, N, H), jnp.float32),
        mesh=_MESH,
        compiler_params=_SC_PARAMS,
        scratch_types=[
            pltpu.VMEM((EPW,), jnp.int32),
            pltpu.VMEM((SCH, H), jnp.float32),
            pltpu.VMEM((SPB, H), jnp.float32),
            pltpu.SemaphoreType.DMA,
            pltpu.VMEM_SHARED((N, H), jnp.float32),
        ],
    )
    return f(edge_index)


def _agg_body(hp_hbm, edge_hbm, out_hbm, src_v, dst_v, rows_v,
              sem_g0, sem_g1, sem_g2, sem_s0, sem_s1, sem_s2, acc):
    cid = lax.axis_index("c")
    sid = lax.axis_index("s")
    wid = sid * NC + cid

    ih = pltpu.async_copy(edge_hbm.at[0, pl.ds(wid * EPW, EPW)], src_v,
                          sem_s0)
    dh = pltpu.async_copy(edge_hbm.at[1, pl.ds(wid * EPW, EPW)], dst_v,
                          sem_s1)

    @pl.when(sid < NSTR)
    def _():
        pltpu.sync_copy(hp_hbm.at[pl.ds(sid * SPB, SPB)],
                        acc.at[pl.ds(sid * SPB, SPB)])
    ih.wait()

    NBUF = 3
    sem_g = (sem_g0, sem_g1, sem_g2)
    sem_s = (sem_s0, sem_s1, sem_s2)
    gh = [None] * NGC
    sh = [[None] * NSC for _ in range(NGC)]
    gh[0] = pltpu.async_copy(hp_hbm.at[src_v.at[pl.ds(0, GCHUNK)]],
                             rows_v.at[0], sem_g0)
    gh[1] = pltpu.async_copy(hp_hbm.at[src_v.at[pl.ds(GCHUNK, GCHUNK)]],
                             rows_v.at[1], sem_g1)
    dh.wait()
    plsc.subcore_barrier()

    for c in range(NGC):
        b = c % NBUF
        gh[c].wait()
        if c + 2 < NGC:
            if c >= 1:
                for k in range(NSC):
                    sh[c - 1][k].wait()
            nb = (c + 2) % NBUF
            gh[c + 2] = pltpu.async_copy(
                hp_hbm.at[src_v.at[pl.ds((c + 2) * GCHUNK, GCHUNK)]],
                rows_v.at[nb], sem_g[nb])
        for k in range(NSC):
            sh[c][k] = pltpu.async_copy(
                rows_v.at[b, pl.ds(k * SCH, SCH)],
                acc.at[dst_v.at[pl.ds((c * NSC + k) * SCH, SCH)]],
                sem_s[b], add=True)
    for k in range(NSC):
        sh[NGC - 2][k].wait()
        sh[NGC - 1][k].wait()

    plsc.subcore_barrier()

    @pl.when(sid < NSTR)
    def _():
        pltpu.sync_copy(acc.at[pl.ds(sid * SPB, SPB)],
                        out_hbm.at[cid, pl.ds(sid * SPB, SPB)])


def _agg_call(hp, edge_index):
    f = pl.kernel(
        _agg_body,
        out_type=jax.ShapeDtypeStruct((NC, N, H), jnp.float32),
        mesh=_MESH,
        compiler_params=_SC_PARAMS,
        scratch_types=[
            pltpu.VMEM((EPW,), jnp.int32),
            pltpu.VMEM((EPW,), jnp.int32),
            pltpu.VMEM((3, GCHUNK, H), jnp.float32),
            pltpu.SemaphoreType.DMA,
            pltpu.SemaphoreType.DMA,
            pltpu.SemaphoreType.DMA,
            pltpu.SemaphoreType.DMA,
            pltpu.SemaphoreType.DMA,
            pltpu.SemaphoreType.DMA,
            pltpu.VMEM_SHARED((N, H), jnp.float32),
        ],
    )
    return f(hp, edge_index)


NP8 = N // 8


def _mm0_kernel(x_ref, w_ref, o_ref):
    o_ref[...] = jnp.dot(x_ref[...], w_ref[...],
                         preferred_element_type=jnp.float32)


def _prep_kernel(degp_ref, h0_ref, dinv_ref, hp_ref):
    deg = degp_ref[0] + degp_ref[1] + 1.0
    dinv = lax.rsqrt(deg)
    dinv_ref[...] = dinv
    hp_ref[...] = h0_ref[...] * dinv


def _combine_kernel(p_ref, hp_ref, dinv_ref, b_ref, w_ref, o_ref):
    s = p_ref[0] + p_ref[1] - hp_ref[...]
    o = s * dinv_ref[...] + b_ref[...]
    a = jnp.where(o >= 0.0, o, 0.01 * o)
    o_ref[...] = jnp.dot(a, w_ref[...],
                         preferred_element_type=jnp.float32) * dinv_ref[...]


def _logits_kernel(p_ref, hp_ref, dinv_ref, b_ref, o_ref):
    s = p_ref[0] + p_ref[1] - hp_ref[...]
    o_ref[...] = s * dinv_ref[...] + b_ref[...]


def _softmax_kernel(x_ref, o_ref):
    o_ref[...] = jax.nn.softmax(x_ref[...], axis=1)


def kernel(x, edge_index, W0, b0, W1, b1, W2, b2, W3, b3):
    f32 = jnp.float32
    sds = jax.ShapeDtypeStruct
    eye8 = jnp.eye(8, dtype=f32)

    h0p = pl.pallas_call(_mm0_kernel, out_shape=sds((NP8, 128), f32))(
        x.reshape(NP8, 8 * F_IN), jnp.kron(eye8, W0))
    degp = _deg_call(edge_index).reshape(NC, NP8, 128)

    dinv, hp = pl.pallas_call(
        _prep_kernel, out_shape=(sds((NP8, 128), f32), sds((NP8, 128), f32))
    )(degp, h0p)

    combine = pl.pallas_call(_combine_kernel, out_shape=sds((NP8, 128), f32))
    for W, b in ((W1, b0), (W2, b1), (W3, b2)):
        p = _agg_call(hp.reshape(N, H), edge_index).reshape(NC, NP8, 128)
        hp = combine(p, hp, dinv, jnp.tile(b, 8).reshape(1, 128),
                     jnp.kron(eye8, W))

    p = _agg_call(hp.reshape(N, H), edge_index).reshape(NC, NP8, 128)
    logits = pl.pallas_call(_logits_kernel, out_shape=sds((NP8, 128), f32))(
        p, hp, dinv, jnp.tile(b3, 8).reshape(1, 128))
    out = pl.pallas_call(_softmax_kernel, out_shape=sds((N, H), f32))(
        logits.reshape(N, H))
    return out


# speedup vs baseline: 89.2249x; 1.0741x over previous
import jax
import jax.numpy as jnp
from jax import lax
from jax.experimental import pallas as pl
from jax.experimental.pallas import tpu as pltpu
from jax.experimental.pallas import tpu_sc as plsc

N = 10000
E = 320000
F_IN = 128
H = 16

NC = 2
NS = 16
NW = NC * NS
EPW = E // NW
GCHUNK = 2000
SCH = 1000
NGC = EPW // GCHUNK
NSC = GCHUNK // SCH
DPW = EPW // SCH
SPB = 1000
NSTR = N // SPB

_MESH = plsc.VectorSubcoreMesh(core_axis_name="c", subcore_axis_name="s")
_SC_PARAMS = pltpu.CompilerParams(use_tc_tiling_on_sc=False)


def _deg_body(edge_hbm, out_hbm, dst_v, ones_v, zbuf_v, sem_s, acc):
    cid = lax.axis_index("c")
    sid = lax.axis_index("s")
    wid = sid * NC + cid

    dh = pltpu.async_copy(edge_hbm.at[1, pl.ds(wid * EPW, EPW)], dst_v,
                          sem_s)

    @pl.loop(0, SCH)
    def _(j):
        ones_v.at[j][...] = jnp.full((H,), 1.0, jnp.float32)

    @pl.loop(0, SPB)
    def _(j):
        zbuf_v.at[j][...] = jnp.zeros((H,), jnp.float32)

    @pl.when(sid < NSTR)
    def _():
        pltpu.sync_copy(zbuf_v, acc.at[pl.ds(sid * SPB, SPB)])
    plsc.subcore_barrier()

    dh.wait()

    handles = []
    for j in range(DPW):
        handles.append(
            pltpu.async_copy(ones_v, acc.at[dst_v.at[pl.ds(j * SCH, SCH)]],
                             sem_s, add=True))
        if j >= 4:
            handles[j - 4].wait()
    for h in handles[DPW - 4:]:
        h.wait()

    plsc.subcore_barrier()

    @pl.when(sid < NSTR)
    def _():
        pltpu.sync_copy(acc.at[pl.ds(sid * SPB, SPB)],
                        out_hbm.at[cid, pl.ds(sid * SPB, SPB)])


def _deg_call(edge_index):
    f = pl.kernel(
        _deg_body,
        out_type=jax.ShapeDtypeStruct((NC, N, H), jnp.float32),
        mesh=_MESH,
        compiler_params=_SC_PARAMS,
        scratch_types=[
            pltpu.VMEM((EPW,), jnp.int32),
            pltpu.VMEM((SCH, H), jnp.float32),
            pltpu.VMEM((SPB, H), jnp.float32),
            pltpu.SemaphoreType.DMA,
            pltpu.VMEM_SHARED((N, H), jnp.float32),
        ],
    )
    return f(edge_index)


def _agg_body(hp_hbm, edge_hbm, out_hbm, src_v, dst_v, rows_v,
              sem_g0, sem_g1, sem_g2, sem_s0, sem_s1, sem_s2, acc):
    cid = lax.axis_index("c")
    sid = lax.axis_index("s")
    wid = sid * NC + cid

    ih = pltpu.async_copy(edge_hbm.at[0, pl.ds(wid * EPW, EPW)], src_v,
                          sem_s0)
    dh = pltpu.async_copy(edge_hbm.at[1, pl.ds(wid * EPW, EPW)], dst_v,
                          sem_s1)

    @pl.when(sid < NSTR)
    def _():
        pltpu.sync_copy(hp_hbm.at[pl.ds(sid * SPB, SPB)],
                        acc.at[pl.ds(sid * SPB, SPB)])
    ih.wait()

    NBUF = 3
    sem_g = (sem_g0, sem_g1, sem_g2)
    sem_s = (sem_s0, sem_s1, sem_s2)
    gh = [None] * NGC
    sh = [[None] * NSC for _ in range(NGC)]
    gh[0] = pltpu.async_copy(hp_hbm.at[src_v.at[pl.ds(0, GCHUNK)]],
                             rows_v.at[0], sem_g0)
    gh[1] = pltpu.async_copy(hp_hbm.at[src_v.at[pl.ds(GCHUNK, GCHUNK)]],
                             rows_v.at[1], sem_g1)
    dh.wait()
    plsc.subcore_barrier()

    for c in range(NGC):
        b = c % NBUF
        gh[c].wait()
        if c + 2 < NGC:
            if c >= 1:
                for k in range(NSC):
                    sh[c - 1][k].wait()
            nb = (c + 2) % NBUF
            gh[c + 2] = pltpu.async_copy(
                hp_hbm.at[src_v.at[pl.ds((c + 2) * GCHUNK, GCHUNK)]],
                rows_v.at[nb], sem_g[nb])
        for k in range(NSC):
            sh[c][k] = pltpu.async_copy(
                rows_v.at[b, pl.ds(k * SCH, SCH)],
                acc.at[dst_v.at[pl.ds((c * NSC + k) * SCH, SCH)]],
                sem_s[b], add=True)
    for k in range(NSC):
        sh[NGC - 2][k].wait()
        sh[NGC - 1][k].wait()

    plsc.subcore_barrier()

    @pl.when(sid < NSTR)
    def _():
        pltpu.sync_copy(acc.at[pl.ds(sid * SPB, SPB)],
                        out_hbm.at[cid, pl.ds(sid * SPB, SPB)])


def _agg_call(hp, edge_index):
    f = pl.kernel(
        _agg_body,
        out_type=jax.ShapeDtypeStruct((NC, N, H), jnp.float32),
        mesh=_MESH,
        compiler_params=_SC_PARAMS,
        scratch_types=[
            pltpu.VMEM((EPW,), jnp.int32),
            pltpu.VMEM((EPW,), jnp.int32),
            pltpu.VMEM((3, GCHUNK, H), jnp.float32),
            pltpu.SemaphoreType.DMA,
            pltpu.SemaphoreType.DMA,
            pltpu.SemaphoreType.DMA,
            pltpu.SemaphoreType.DMA,
            pltpu.SemaphoreType.DMA,
            pltpu.SemaphoreType.DMA,
            pltpu.VMEM_SHARED((N, H), jnp.float32),
        ],
    )
    return f(hp, edge_index)


NP8 = N // 8


def _mm0_kernel(x_ref, w_ref, o_ref):
    o_ref[...] = jnp.dot(x_ref[...], w_ref[...],
                         preferred_element_type=jnp.float32)


def _prep_kernel(degp_ref, h0_ref, dinv_ref, hp_ref):
    deg = degp_ref[0] + degp_ref[1] + 1.0
    dinv = lax.rsqrt(deg)
    dinv_ref[...] = dinv
    hp_ref[...] = h0_ref[...] * dinv


def _combine_kernel(p_ref, hp_ref, dinv_ref, b_ref, w_ref, o_ref):
    s = p_ref[0] + p_ref[1] - hp_ref[...]
    o = s * dinv_ref[...] + b_ref[...]
    a = jnp.where(o >= 0.0, o, 0.01 * o)
    o_ref[...] = jnp.dot(a, w_ref[...],
                         preferred_element_type=jnp.float32) * dinv_ref[...]


def _final_kernel(p_ref, hp_ref, dinv_ref, b_ref, g_ref, o_ref):
    s = p_ref[0] + p_ref[1] - hp_ref[...]
    o = s * dinv_ref[...] + b_ref[...]
    c = jnp.max(o, axis=1, keepdims=True)
    e = jnp.exp(o - c)
    den = jnp.dot(e, g_ref[...], preferred_element_type=jnp.float32)
    o_ref[...] = e / den


def kernel(x, edge_index, W0, b0, W1, b1, W2, b2, W3, b3):
    f32 = jnp.float32
    sds = jax.ShapeDtypeStruct
    eye8 = jnp.eye(8, dtype=f32)

    h0p = pl.pallas_call(_mm0_kernel, out_shape=sds((NP8, 128), f32))(
        x.reshape(NP8, 8 * F_IN), jnp.kron(eye8, W0))
    degp = _deg_call(edge_index).reshape(NC, NP8, 128)

    dinv, hp = pl.pallas_call(
        _prep_kernel, out_shape=(sds((NP8, 128), f32), sds((NP8, 128), f32))
    )(degp, h0p)

    combine = pl.pallas_call(_combine_kernel, out_shape=sds((NP8, 128), f32))
    for W, b in ((W1, b0), (W2, b1), (W3, b2)):
        p = _agg_call(hp.reshape(N, H), edge_index).reshape(NC, NP8, 128)
        hp = combine(p, hp, dinv, jnp.tile(b, 8).reshape(1, 128),
                     jnp.kron(eye8, W))

    p = _agg_call(hp.reshape(N, H), edge_index).reshape(NC, NP8, 128)
    out_p = pl.pallas_call(_final_kernel, out_shape=sds((NP8, 128), f32))(
        p, hp, dinv, jnp.tile(b3, 8).reshape(1, 128),
        jnp.kron(eye8, jnp.ones((H, H), f32)))
    return out_p.reshape(N, H)
